# baseline (speedup 1.0000x reference)
import functools

import jax
import jax.numpy as jnp
from jax import lax
from jax.experimental import pallas as pl
from jax.experimental.pallas import tpu as pltpu
from jax.experimental.pallas import tpu_sc as plsc

N = 10000
E = 320000
D = 128
A = 1024
NC = 2
NS = 16
NW = NC * NS
L = 16

N_PAD = 10240
SB = 2000
EH = E // NS
EW = E // NW
NPW = N_PAD // NW
APC = A // NS
CAP = SB + 4 * L
ZR = 32

_MAGIC = 0x5F3759DF


def _qrsqrt(v):
    i = plsc.bitcast(v, jnp.int32)
    i = jnp.full((L,), _MAGIC, jnp.int32) - lax.shift_right_logical(i, 1)
    y = plsc.bitcast(i, jnp.float32)
    for _ in range(3):
        y = y * (1.5 - 0.5 * v * y * y)
    return y


def _sc_body(src_h, dst_h, am_h, x_h, part_h,
             tabl, amv, st_src, st_dst, csrc, cdst, cw,
             rows, rows2, zrows, dsum, dtmp,
             acc_sh, stage_sh, dinv_sh, sem, sem2):
    cid = lax.axis_index("c")
    sid = lax.axis_index("s")
    wid = sid * NC + cid

    ones16 = jnp.ones((L,), jnp.float32)
    zero16 = jnp.zeros((L,), jnp.float32)

    def _zero_tabl(i, _):
        tabl[pl.ds(i * L, L)] = zero16
        return 0
    lax.fori_loop(0, N_PAD // L, _zero_tabl, 0)

    def _zero_zrows(r, _):
        for k in range(D // L):
            zrows[r, pl.ds(k * L, L)] = zero16
        return 0
    lax.fori_loop(0, ZR, _zero_zrows, 0)

    rows_per_tile = N_PAD // NS
    for k in range(rows_per_tile // ZR):
        pltpu.sync_copy(zrows, acc_sh.at[pl.ds(sid * rows_per_tile + k * ZR, ZR), :])

    def _hist_block(b, _):
        pltpu.sync_copy(dst_h.at[pl.ds(sid * EH + b * SB, SB)], st_dst)

        def _chunk(i, _):
            d16 = st_dst[pl.ds(i * L, L)]
            plsc.addupdate_scatter(tabl, [d16], ones16)
            return 0
        lax.fori_loop(0, SB // L, _chunk, 0)
        return 0
    lax.fori_loop(0, EH // SB, _hist_block, 0)

    pltpu.sync_copy(tabl, stage_sh.at[sid])
    plsc.subcore_barrier()

    rpt = N_PAD // NS
    base = sid * rpt
    pltpu.sync_copy(stage_sh.at[0, pl.ds(base, rpt)], dsum)
    for j in range(1, NS):
        pltpu.sync_copy(stage_sh.at[j, pl.ds(base, rpt)], dtmp)

        def _acc(i, _):
            dsum[pl.ds(i * L, L)] = dsum[pl.ds(i * L, L)] + dtmp[pl.ds(i * L, L)]
            return 0
        lax.fori_loop(0, rpt // L, _acc, 0)

    def _rsq(i, _):
        v = dsum[pl.ds(i * L, L)] + 1.0
        dsum[pl.ds(i * L, L)] = _qrsqrt(v)
        return 0
    lax.fori_loop(0, rpt // L, _rsq, 0)
    pltpu.sync_copy(dsum, dinv_sh.at[pl.ds(base, rpt)])
    plsc.subcore_barrier()
    pltpu.sync_copy(dinv_sh, tabl)

    pltpu.sync_copy(am_h, amv)

    def _flag(i, _):
        m16 = amv[pl.ds(i * L, L)]
        g = plsc.load_gather(tabl, [m16])
        plsc.store_scatter(tabl, [m16], -jnp.abs(g))
        return 0
    lax.fori_loop(0, A // L, _flag, 0)

    def _issue(o, buf, sm):
        s16 = csrc[pl.ds(o, L)]
        pltpu.async_copy(x_h.at[s16], buf, sm)

    def _wait(buf, sm):
        pltpu.make_async_copy(x_h.at[pl.ds(0, L)], buf, sm).wait()

    def _scale_scatter(o, buf):
        d16 = cdst[pl.ds(o, L)]
        w16 = cw[pl.ds(o, L)]
        for jj in range(L):
            wsp = jnp.take_along_axis(
                w16, jnp.full((L,), jj, jnp.int32), axis=0,
                mode="promise_in_bounds")
            for k in range(D // L):
                buf[jj, pl.ds(k * L, L)] = buf[jj, pl.ds(k * L, L)] * wsp
        pltpu.sync_copy(buf, acc_sh.at[d16], add=True)

    def _consume(off):
        for p in range(4):
            csrc[pl.ds(off + p * L, L)] = jnp.zeros((L,), jnp.int32)
            cdst[pl.ds(off + p * L, L)] = jnp.full((L,), N, jnp.int32)
            cw[pl.ds(off + p * L, L)] = zero16
        npairs = jnp.maximum(lax.shift_right_logical(off + (2 * L - 1), 5), 1)

        _issue(0, rows, sem)

        def _gss(j, _):
            o = j * 2 * L
            _wait(rows, sem)
            _issue(o + L, rows2, sem2)
            _scale_scatter(o, rows)
            _wait(rows2, sem2)
            _issue(o + 2 * L, rows, sem)
            _scale_scatter(o + L, rows2)
            return 0
        lax.fori_loop(0, npairs, _gss, 0)
        _wait(rows, sem)

    ebase = wid * EW

    def _edge_block(b, _):
        pltpu.sync_copy(src_h.at[pl.ds(ebase + b * SB, SB)], st_src)
        pltpu.sync_copy(dst_h.at[pl.ds(ebase + b * SB, SB)], st_dst)

        def _chunk(i, off):
            s16 = st_src[pl.ds(i * L, L)]
            d16 = st_dst[pl.ds(i * L, L)]
            td = plsc.load_gather(tabl, [d16])
            msk = td < 0.0
            ts = plsc.load_gather(tabl, [s16])
            w16 = jnp.abs(ts) * jnp.abs(td)
            plsc.store_compressed(csrc.at[pl.ds(off, L)], s16, mask=msk)
            plsc.store_compressed(cdst.at[pl.ds(off, L)], d16, mask=msk)
            plsc.store_compressed(cw.at[pl.ds(off, L)], w16, mask=msk)
            return off + jnp.sum(msk.astype(jnp.int32))

        off = lax.fori_loop(0, SB // L, _chunk, jnp.int32(0))
        _consume(off)
        return 0

    lax.fori_loop(0, EW // SB, _edge_block, 0)

    nbase = wid * NPW

    def _selfloop(i, off):
        b = nbase + i * L
        t16 = tabl[pl.ds(b, L)]
        msk = t16 < 0.0
        dv = jnp.abs(t16)
        n16 = lax.iota(jnp.int32, L) + b
        plsc.store_compressed(csrc.at[pl.ds(off, L)], n16, mask=msk)
        plsc.store_compressed(cdst.at[pl.ds(off, L)], n16, mask=msk)
        plsc.store_compressed(cw.at[pl.ds(off, L)], dv * dv, mask=msk)
        return off + jnp.sum(msk.astype(jnp.int32))

    off = lax.fori_loop(0, NPW // L, _selfloop, jnp.int32(0))
    _consume(off)

    plsc.subcore_barrier()

    abase = sid * APC
    for t in range(APC // L):
        m16 = amv[pl.ds(abase + t * L, L)]
        pltpu.async_copy(acc_sh.at[m16], rows, sem).wait()
        pltpu.sync_copy(rows, part_h.at[cid, pl.ds(abase + t * L, L), :])


@functools.cache
def _sc_gather_fn():
  return functools.partial(
    pl.kernel,
    out_type=jax.ShapeDtypeStruct((NC, A, D), jnp.float32),
    mesh=plsc.VectorSubcoreMesh(core_axis_name="c", subcore_axis_name="s"),
    scratch_types=[
        pltpu.VMEM((N_PAD,), jnp.float32),
        pltpu.VMEM((A,), jnp.int32),
        pltpu.VMEM((SB,), jnp.int32),
        pltpu.VMEM((SB,), jnp.int32),
        pltpu.VMEM((CAP,), jnp.int32),
        pltpu.VMEM((CAP,), jnp.int32),
        pltpu.VMEM((CAP,), jnp.float32),
        pltpu.VMEM((L, D), jnp.float32),
        pltpu.VMEM((L, D), jnp.float32),
        pltpu.VMEM((ZR, D), jnp.float32),
        pltpu.VMEM((N_PAD // NS,), jnp.float32),
        pltpu.VMEM((N_PAD // NS,), jnp.float32),
        pltpu.VMEM_SHARED((N_PAD, D), jnp.float32),
        pltpu.VMEM_SHARED((NS, N_PAD), jnp.float32),
        pltpu.VMEM_SHARED((N_PAD,), jnp.float32),
        pltpu.SemaphoreType.DMA,
        pltpu.SemaphoreType.DMA,
    ],
    compiler_params=pltpu.CompilerParams(needs_layout_passes=False),
  )(_sc_body)


def _ln(v, g, b, eps=1e-5):
    mu = jnp.mean(v, axis=-1, keepdims=True)
    var = jnp.mean((v - mu) ** 2, axis=-1, keepdims=True)
    return (v - mu) * lax.rsqrt(var + eps) * g + b


def _head_body(part_ref, action_ref, Wg_ref, bg_ref, W1_ref, b1_ref, g1_ref,
               be1_ref, W2_ref, b2_ref, g2_ref, be2_ref, Wa_ref, ba_ref,
               Wq_ref, bq_ref, q_ref):
    agg = part_ref[0] + part_ref[1]
    h = jnp.dot(agg, Wg_ref[...], preferred_element_type=jnp.float32, precision=lax.Precision.HIGHEST) + bg_ref[...]
    h = jnp.maximum(h, 0.0)
    sv = jnp.dot(h, W1_ref[...], preferred_element_type=jnp.float32, precision=lax.Precision.HIGHEST) + b1_ref[...]
    sv = _ln(sv, g1_ref[...], be1_ref[...])
    sv = jnp.maximum(sv, 0.0)
    sv = jnp.dot(sv, W2_ref[...], preferred_element_type=jnp.float32, precision=lax.Precision.HIGHEST) + b2_ref[...]
    sv = _ln(sv, g2_ref[...], be2_ref[...])
    av = jnp.dot(action_ref[...], Wa_ref[...], preferred_element_type=jnp.float32, precision=lax.Precision.HIGHEST) + ba_ref[...]
    sav = jnp.maximum(sv + av, 0.0)
    q_ref[...] = jnp.dot(sav, Wq_ref[...], preferred_element_type=jnp.float32, precision=lax.Precision.HIGHEST) + bq_ref[...]


def kernel(x, edge_index, action, agent_mask, W_gcn, b_gcn, W1, b1, g1, be1,
           W2, b2, g2, be2, Wa, ba, Wq, bq):
    src = edge_index[0]
    dst = edge_index[1]
    part = _sc_gather_fn()(src, dst, agent_mask, x)
    q = pl.pallas_call(
        _head_body,
        out_shape=jax.ShapeDtypeStruct((A, 1), jnp.float32),
    )(part, action, W_gcn, b_gcn.reshape(1, -1), W1, b1.reshape(1, -1),
      g1.reshape(1, -1), be1.reshape(1, -1), W2, b2.reshape(1, -1),
      g2.reshape(1, -1), be2.reshape(1, -1), Wa, ba.reshape(1, -1),
      Wq, bq.reshape(1, 1))
    return q


# speedup vs baseline: 1.5115x; 1.5115x over previous
import functools

import jax
import jax.numpy as jnp
from jax import lax
from jax.experimental import pallas as pl
from jax.experimental.pallas import tpu as pltpu
from jax.experimental.pallas import tpu_sc as plsc

N = 10000
E = 320000
D = 128
A = 1024
NC = 2
NS = 16
NW = NC * NS
L = 16

N_PAD = 10240
SB = 2000
EH = E // NS
EW = E // NW
NPW = N_PAD // NW
APC = A // NS
CAP = SB + 4 * L
ZR = 32

_MAGIC = 0x5F3759DF


def _qrsqrt(v):
    i = plsc.bitcast(v, jnp.int32)
    i = jnp.full((L,), _MAGIC, jnp.int32) - lax.shift_right_logical(i, 1)
    y = plsc.bitcast(i, jnp.float32)
    for _ in range(3):
        y = y * (1.5 - 0.5 * v * y * y)
    return y


def _sc_body(src_h, dst_h, am_h, x_h, part_h,
             tabl, amv, st_src, st_dst, csrc, cdst, cw,
             rows, rows2, zrows, dsum, dtmp,
             acc_sh, stage_sh, dinv_sh, sem, sem2):
    cid = lax.axis_index("c")
    sid = lax.axis_index("s")
    wid = sid * NC + cid

    ones16 = jnp.ones((L,), jnp.float32)
    zero16 = jnp.zeros((L,), jnp.float32)

    def _zero_tabl(i, _):
        tabl[pl.ds(i * L, L)] = zero16
        return 0
    lax.fori_loop(0, N_PAD // L, _zero_tabl, 0)

    def _zero_zrows(r, _):
        for k in range(D // L):
            zrows[r, pl.ds(k * L, L)] = zero16
        return 0
    lax.fori_loop(0, ZR, _zero_zrows, 0)

    rows_per_tile = N_PAD // NS
    for k in range(rows_per_tile // ZR):
        pltpu.sync_copy(zrows, acc_sh.at[pl.ds(sid * rows_per_tile + k * ZR, ZR), :])

    def _hist_block(b, _):
        pltpu.sync_copy(dst_h.at[pl.ds(sid * EH + b * SB, SB)], st_dst)

        def _chunk(i, _):
            d16 = st_dst[pl.ds(i * L, L)]
            plsc.addupdate_scatter(tabl, [d16], ones16)
            return 0
        lax.fori_loop(0, SB // L, _chunk, 0)
        return 0
    lax.fori_loop(0, EH // SB, _hist_block, 0)

    pltpu.sync_copy(tabl, stage_sh.at[sid])
    plsc.subcore_barrier()

    rpt = N_PAD // NS
    base = sid * rpt
    pltpu.sync_copy(stage_sh.at[0, pl.ds(base, rpt)], dsum)
    for j in range(1, NS):
        pltpu.sync_copy(stage_sh.at[j, pl.ds(base, rpt)], dtmp)

        def _acc(i, _):
            dsum[pl.ds(i * L, L)] = dsum[pl.ds(i * L, L)] + dtmp[pl.ds(i * L, L)]
            return 0
        lax.fori_loop(0, rpt // L, _acc, 0)

    def _rsq(i, _):
        v = dsum[pl.ds(i * L, L)] + 1.0
        dsum[pl.ds(i * L, L)] = _qrsqrt(v)
        return 0
    lax.fori_loop(0, rpt // L, _rsq, 0)
    pltpu.sync_copy(dsum, dinv_sh.at[pl.ds(base, rpt)])
    plsc.subcore_barrier()
    pltpu.sync_copy(dinv_sh, tabl)

    pltpu.sync_copy(am_h, amv)

    def _flag(i, _):
        m16 = amv[pl.ds(i * L, L)]
        g = plsc.load_gather(tabl, [m16])
        plsc.store_scatter(tabl, [m16], -jnp.abs(g))
        return 0
    lax.fori_loop(0, A // L, _flag, 0)

    def _consume(off):
        for p in range(2):
            csrc[pl.ds(off + p * L, L)] = jnp.zeros((L,), jnp.int32)
            cdst[pl.ds(off + p * L, L)] = jnp.full((L,), N, jnp.int32)
            cw[pl.ds(off + p * L, L)] = zero16
        npairs = lax.shift_right_logical(off + (2 * L - 1), 5)

        def _one(o, buf):
            s16 = csrc[pl.ds(o, L)]
            d16 = cdst[pl.ds(o, L)]
            w16 = cw[pl.ds(o, L)]
            pltpu.async_copy(x_h.at[s16], buf, sem).wait()
            for jj in range(L):
                wsp = jnp.take_along_axis(
                    w16, jnp.full((L,), jj, jnp.int32), axis=0,
                    mode="promise_in_bounds")
                for k in range(D // L):
                    buf[jj, pl.ds(k * L, L)] = buf[jj, pl.ds(k * L, L)] * wsp
            pltpu.sync_copy(buf, acc_sh.at[d16], add=True)

        def _gss(j, _):
            _one(j * 2 * L, rows)
            _one(j * 2 * L + L, rows2)
            return 0
        lax.fori_loop(0, npairs, _gss, 0)

    ebase = wid * EW

    def _edge_block(b, _):
        pltpu.sync_copy(src_h.at[pl.ds(ebase + b * SB, SB)], st_src)
        pltpu.sync_copy(dst_h.at[pl.ds(ebase + b * SB, SB)], st_dst)

        def _chunk(i, off):
            s16 = st_src[pl.ds(i * L, L)]
            d16 = st_dst[pl.ds(i * L, L)]
            td = plsc.load_gather(tabl, [d16])
            msk = td < 0.0
            ts = plsc.load_gather(tabl, [s16])
            w16 = jnp.abs(ts) * jnp.abs(td)
            plsc.store_compressed(csrc.at[pl.ds(off, L)], s16, mask=msk)
            plsc.store_compressed(cdst.at[pl.ds(off, L)], d16, mask=msk)
            plsc.store_compressed(cw.at[pl.ds(off, L)], w16, mask=msk)
            return off + jnp.sum(msk.astype(jnp.int32))

        off = lax.fori_loop(0, SB // L, _chunk, jnp.int32(0))
        _consume(off)
        return 0

    lax.fori_loop(0, EW // SB, _edge_block, 0)

    nbase = wid * NPW

    def _selfloop(i, off):
        b = nbase + i * L
        t16 = tabl[pl.ds(b, L)]
        msk = t16 < 0.0
        dv = jnp.abs(t16)
        n16 = lax.iota(jnp.int32, L) + b
        plsc.store_compressed(csrc.at[pl.ds(off, L)], n16, mask=msk)
        plsc.store_compressed(cdst.at[pl.ds(off, L)], n16, mask=msk)
        plsc.store_compressed(cw.at[pl.ds(off, L)], dv * dv, mask=msk)
        return off + jnp.sum(msk.astype(jnp.int32))

    off = lax.fori_loop(0, NPW // L, _selfloop, jnp.int32(0))
    _consume(off)

    plsc.subcore_barrier()

    abase = sid * APC
    for t in range(APC // L):
        m16 = amv[pl.ds(abase + t * L, L)]
        pltpu.async_copy(acc_sh.at[m16], rows, sem).wait()
        pltpu.sync_copy(rows, part_h.at[cid, pl.ds(abase + t * L, L), :])


@functools.cache
def _sc_gather_fn():
  return functools.partial(
    pl.kernel,
    out_type=jax.ShapeDtypeStruct((NC, A, D), jnp.float32),
    mesh=plsc.VectorSubcoreMesh(core_axis_name="c", subcore_axis_name="s"),
    scratch_types=[
        pltpu.VMEM((N_PAD,), jnp.float32),
        pltpu.VMEM((A,), jnp.int32),
        pltpu.VMEM((SB,), jnp.int32),
        pltpu.VMEM((SB,), jnp.int32),
        pltpu.VMEM((CAP,), jnp.int32),
        pltpu.VMEM((CAP,), jnp.int32),
        pltpu.VMEM((CAP,), jnp.float32),
        pltpu.VMEM((L, D), jnp.float32),
        pltpu.VMEM((L, D), jnp.float32),
        pltpu.VMEM((ZR, D), jnp.float32),
        pltpu.VMEM((N_PAD // NS,), jnp.float32),
        pltpu.VMEM((N_PAD // NS,), jnp.float32),
        pltpu.VMEM_SHARED((N_PAD, D), jnp.float32),
        pltpu.VMEM_SHARED((NS, N_PAD), jnp.float32),
        pltpu.VMEM_SHARED((N_PAD,), jnp.float32),
        pltpu.SemaphoreType.DMA,
        pltpu.SemaphoreType.DMA,
    ],
    compiler_params=pltpu.CompilerParams(needs_layout_passes=False),
  )(_sc_body)


def _ln(v, g, b, eps=1e-5):
    mu = jnp.mean(v, axis=-1, keepdims=True)
    var = jnp.mean((v - mu) ** 2, axis=-1, keepdims=True)
    return (v - mu) * lax.rsqrt(var + eps) * g + b


def _head_body(part_ref, action_ref, Wg_ref, bg_ref, W1_ref, b1_ref, g1_ref,
               be1_ref, W2_ref, b2_ref, g2_ref, be2_ref, Wa_ref, ba_ref,
               Wq_ref, bq_ref, q_ref):
    agg = part_ref[0] + part_ref[1]
    h = jnp.dot(agg, Wg_ref[...], preferred_element_type=jnp.float32, precision=lax.Precision.HIGHEST) + bg_ref[...]
    h = jnp.maximum(h, 0.0)
    sv = jnp.dot(h, W1_ref[...], preferred_element_type=jnp.float32, precision=lax.Precision.HIGHEST) + b1_ref[...]
    sv = _ln(sv, g1_ref[...], be1_ref[...])
    sv = jnp.maximum(sv, 0.0)
    sv = jnp.dot(sv, W2_ref[...], preferred_element_type=jnp.float32, precision=lax.Precision.HIGHEST) + b2_ref[...]
    sv = _ln(sv, g2_ref[...], be2_ref[...])
    av = jnp.dot(action_ref[...], Wa_ref[...], preferred_element_type=jnp.float32, precision=lax.Precision.HIGHEST) + ba_ref[...]
    sav = jnp.maximum(sv + av, 0.0)
    q_ref[...] = jnp.dot(sav, Wq_ref[...], preferred_element_type=jnp.float32, precision=lax.Precision.HIGHEST) + bq_ref[...]


def kernel(x, edge_index, action, agent_mask, W_gcn, b_gcn, W1, b1, g1, be1,
           W2, b2, g2, be2, Wa, ba, Wq, bq):
    src = edge_index[0]
    dst = edge_index[1]
    part = _sc_gather_fn()(src, dst, agent_mask, x)
    q = pl.pallas_call(
        _head_body,
        out_shape=jax.ShapeDtypeStruct((A, 1), jnp.float32),
    )(part, action, W_gcn, b_gcn.reshape(1, -1), W1, b1.reshape(1, -1),
      g1.reshape(1, -1), be1.reshape(1, -1), W2, b2.reshape(1, -1),
      g2.reshape(1, -1), be2.reshape(1, -1), Wa, ba.reshape(1, -1),
      Wq, bq.reshape(1, 1))
    return q
